# Initial kernel scaffold; baseline (speedup 1.0000x reference)
import jax
import jax.numpy as jnp
from jax.experimental import pallas as pl


def kernel(entity_emb, user_emb, latent_emb, weight, interact_mat):
    raise NotImplementedError("write your pallas kernel here")


# speedup vs baseline: 1.1007x; 1.1007x over previous
import functools

import jax
import jax.numpy as jnp
from jax.experimental import pallas as pl

BM = 512
BK = 4096


def _agg_kernel(user_ref, latent_ref, weight_ref, interact_ref, entity_ref,
                out_ref, *, nk):
    k = pl.program_id(1)

    part = jnp.dot(interact_ref[...], entity_ref[...],
                   preferred_element_type=jnp.float32)

    @pl.when(k == 0)
    def _init():
        out_ref[...] = part

    @pl.when(k > 0)
    def _acc():
        out_ref[...] += part

    @pl.when(k == nk - 1)
    def _finish():
        score = jnp.dot(user_ref[...], latent_ref[...].T,
                        preferred_element_type=jnp.float32)
        score = jax.nn.softmax(score, axis=1)
        gate = jnp.dot(score, weight_ref[...],
                       preferred_element_type=jnp.float32)
        out_ref[...] *= (1.0 + gate)


@jax.jit
def kernel(entity_emb, user_emb, latent_emb, weight, interact_mat):
    n_users, n_entities = interact_mat.shape
    channel = entity_emb.shape[1]
    nm = n_users // BM
    nk = n_entities // BK

    return pl.pallas_call(
        functools.partial(_agg_kernel, nk=nk),
        grid=(nm, nk),
        in_specs=[
            pl.BlockSpec((BM, channel), lambda m, k: (m, 0)),
            pl.BlockSpec(latent_emb.shape, lambda m, k: (0, 0)),
            pl.BlockSpec(weight.shape, lambda m, k: (0, 0)),
            pl.BlockSpec((BM, BK), lambda m, k: (m, k)),
            pl.BlockSpec((BK, channel), lambda m, k: (k, 0)),
        ],
        out_specs=pl.BlockSpec((BM, channel), lambda m, k: (m, 0)),
        out_shape=jax.ShapeDtypeStruct((n_users, channel), jnp.float32),
    )(user_emb, latent_emb, weight, interact_mat, entity_emb)


# speedup vs baseline: 1.3063x; 1.1868x over previous
import jax
import jax.numpy as jnp
from jax.experimental import pallas as pl

BM = 256


def _agg_kernel(user_ref, latent_ref, weight_ref, interact_ref, entity_ref,
                out_ref):
    agg = jnp.dot(interact_ref[...], entity_ref[...],
                  preferred_element_type=jnp.float32)
    score = jnp.dot(user_ref[...], latent_ref[...].T,
                    preferred_element_type=jnp.float32)
    score = jax.nn.softmax(score, axis=1)
    gate = jnp.dot(score, weight_ref[...],
                   preferred_element_type=jnp.float32)
    out_ref[...] = agg * (1.0 + gate)


@jax.jit
def kernel(entity_emb, user_emb, latent_emb, weight, interact_mat):
    n_users, n_entities = interact_mat.shape
    channel = entity_emb.shape[1]
    nm = n_users // BM

    return pl.pallas_call(
        _agg_kernel,
        grid=(nm,),
        in_specs=[
            pl.BlockSpec((BM, channel), lambda m: (m, 0)),
            pl.BlockSpec(latent_emb.shape, lambda m: (0, 0)),
            pl.BlockSpec(weight.shape, lambda m: (0, 0)),
            pl.BlockSpec((BM, n_entities), lambda m: (m, 0)),
            pl.BlockSpec((n_entities, channel), lambda m: (0, 0)),
        ],
        out_specs=pl.BlockSpec((BM, channel), lambda m: (m, 0)),
        out_shape=jax.ShapeDtypeStruct((n_users, channel), jnp.float32),
    )(user_emb, latent_emb, weight, interact_mat, entity_emb)


# speedup vs baseline: 1.3137x; 1.0057x over previous
import jax
import jax.numpy as jnp
from jax.experimental import pallas as pl

BM = 256


def _agg_kernel(user_ref, latent_ref, weight_ref, interact_ref, entity_ref,
                out_ref):
    agg = jnp.dot(interact_ref[...].astype(jnp.bfloat16),
                  entity_ref[...].astype(jnp.bfloat16),
                  preferred_element_type=jnp.float32)
    score = jnp.dot(user_ref[...], latent_ref[...].T,
                    preferred_element_type=jnp.float32)
    score = jax.nn.softmax(score, axis=1)
    gate = jnp.dot(score, weight_ref[...],
                   preferred_element_type=jnp.float32)
    out_ref[...] = agg * (1.0 + gate)


@jax.jit
def kernel(entity_emb, user_emb, latent_emb, weight, interact_mat):
    n_users, n_entities = interact_mat.shape
    channel = entity_emb.shape[1]
    nm = n_users // BM

    return pl.pallas_call(
        _agg_kernel,
        grid=(nm,),
        in_specs=[
            pl.BlockSpec((BM, channel), lambda m: (m, 0)),
            pl.BlockSpec(latent_emb.shape, lambda m: (0, 0)),
            pl.BlockSpec(weight.shape, lambda m: (0, 0)),
            pl.BlockSpec((BM, n_entities), lambda m: (m, 0)),
            pl.BlockSpec((n_entities, channel), lambda m: (0, 0)),
        ],
        out_specs=pl.BlockSpec((BM, channel), lambda m: (m, 0)),
        out_shape=jax.ShapeDtypeStruct((n_users, channel), jnp.float32),
    )(user_emb, latent_emb, weight, interact_mat, entity_emb)


# speedup vs baseline: 1.3200x; 1.0048x over previous
import jax
import jax.numpy as jnp
from jax.experimental import pallas as pl

BM = 128


def _agg_kernel(user_ref, latent_ref, weight_ref, interact_ref, entity_ref,
                out_ref):
    agg = jnp.dot(interact_ref[...].astype(jnp.bfloat16),
                  entity_ref[...].astype(jnp.bfloat16),
                  preferred_element_type=jnp.float32)
    score = jnp.dot(user_ref[...], latent_ref[...].T,
                    preferred_element_type=jnp.float32)
    score = jax.nn.softmax(score, axis=1)
    gate = jnp.dot(score, weight_ref[...],
                   preferred_element_type=jnp.float32)
    out_ref[...] = agg * (1.0 + gate)


@jax.jit
def kernel(entity_emb, user_emb, latent_emb, weight, interact_mat):
    n_users, n_entities = interact_mat.shape
    channel = entity_emb.shape[1]
    nm = n_users // BM

    return pl.pallas_call(
        _agg_kernel,
        grid=(nm,),
        in_specs=[
            pl.BlockSpec((BM, channel), lambda m: (m, 0)),
            pl.BlockSpec(latent_emb.shape, lambda m: (0, 0)),
            pl.BlockSpec(weight.shape, lambda m: (0, 0)),
            pl.BlockSpec((BM, n_entities), lambda m: (m, 0)),
            pl.BlockSpec((n_entities, channel), lambda m: (0, 0)),
        ],
        out_specs=pl.BlockSpec((BM, channel), lambda m: (m, 0)),
        out_shape=jax.ShapeDtypeStruct((n_users, channel), jnp.float32),
    )(user_emb, latent_emb, weight, interact_mat, entity_emb)
